# Initial kernel scaffold; baseline (speedup 1.0000x reference)
import jax
import jax.numpy as jnp
from jax.experimental import pallas as pl


def kernel(x, edge_index, W0, b0, g0, bt0, W1, b1, g1, bt1, W2, b2, g2, bt2, W3, b3, g3, bt3):
    raise NotImplementedError("write your pallas kernel here")


# speedup vs baseline: 4.9238x; 4.9238x over previous
import functools

import jax
import jax.numpy as jnp
from jax import lax
from jax.experimental import pallas as pl
from jax.experimental.pallas import tpu as pltpu
from jax.experimental.pallas import tpu_sc as plsc

N = 10000
E = 160000
DH = 512
F = 128
NCH = DH // F
EPS = 1e-5
NTILE = 32
EPAD = NTILE * 40 * 128
ACC_R = 10240
SLAB = ACC_R // 16
HSLAB = SLAB // 2

def _deg_body(dst16_hbm, ones_hbm, zeros_hbm, out_hbm, dst_v, ones_v, zb, acc):
    cid = lax.axis_index("c")
    sid = lax.axis_index("s")
    pltpu.sync_copy(dst16_hbm.at[pl.ds(sid, 1)], dst_v)
    pltpu.sync_copy(ones_hbm, ones_v)
    pltpu.sync_copy(zeros_hbm, zb)
    r0 = pl.multiple_of(sid * SLAB, 8)
    for z in range(5):
        pltpu.sync_copy(zb, acc.at[pl.ds(r0 + z * 128, 128)])
    plsc.subcore_barrier()
    for j in range(80):
        pltpu.sync_copy(ones_v, acc.at[dst_v.at[0, j]], add=True)
    plsc.subcore_barrier()

    @pl.when(cid == 0)
    def _():
        for z in range(5):
            pltpu.sync_copy(acc.at[pl.ds(r0 + z * 128, 128)], zb)
            pltpu.sync_copy(zb, out_hbm.at[pl.ds(r0 + z * 128, 128)])


def _agg_body(y_hbm, src_hbm, dst_hbm, zeros_hbm, out_hbm,
              src_v, dst_v, gb0, gb1, acc, sem0, sem1):
    cid = lax.axis_index("c")
    sid = lax.axis_index("s")
    ebase = pl.multiple_of(sid * 10240, 8)
    pltpu.sync_copy(dst_hbm.at[pl.ds(sid, 1)], dst_v)
    r0 = pl.multiple_of(sid * SLAB, 8)
    bufs = (gb0, gb1)
    sems = (sem0, sem1)
    for k in range(NCH // 2):
        chunk = cid + 2 * k
        yv = y_hbm.at[chunk]
        pltpu.sync_copy(zeros_hbm, gb0)
        for z in range(5):
            pltpu.sync_copy(gb0, acc.at[pl.ds(r0 + z * 128, 128)])
        plsc.subcore_barrier()
        for half in range(2):
            pltpu.sync_copy(
                src_hbm.at[pl.ds(ebase + half * 5120, 5120)], src_v)
            d = pltpu.async_copy(yv.at[src_v.at[pl.ds(0, 128)]], gb0, sem0)
            for j in range(40):
                d.wait()
                if j < 39:
                    d = pltpu.async_copy(
                        yv.at[src_v.at[pl.ds((j + 1) * 128, 128)]],
                        bufs[(j + 1) % 2], sems[(j + 1) % 2])
                pltpu.sync_copy(bufs[j % 2],
                                acc.at[dst_v.at[0, half * 40 + j]], add=True)
        plsc.subcore_barrier()
        for z in range(5):
            b = bufs[z % 2]
            pltpu.sync_copy(acc.at[pl.ds(r0 + z * 128, 128)], b)
            pltpu.sync_copy(b, out_hbm.at[chunk, pl.ds(r0 + z * 128, 128)])
        plsc.subcore_barrier()


@functools.cache
def _sc_kernels():
    mesh = plsc.VectorSubcoreMesh(core_axis_name="c", subcore_axis_name="s")
    deg = pl.kernel(
        _deg_body,
        out_type=jax.ShapeDtypeStruct((ACC_R, F), jnp.float32),
        mesh=mesh,
        scratch_types=[
            pltpu.VMEM((1, 80, 128), jnp.int32),
            pltpu.VMEM((128, F), jnp.float32),
            pltpu.VMEM((128, F), jnp.float32),
            pltpu.VMEM_SHARED((ACC_R, F), jnp.float32),
        ],
    )
    agg = pl.kernel(
        _agg_body,
        out_type=jax.ShapeDtypeStruct((NCH, ACC_R, F), jnp.float32),
        mesh=mesh,
        scratch_types=[
            pltpu.VMEM((5120,), jnp.int32),
            pltpu.VMEM((1, 80, 128), jnp.int32),
            pltpu.VMEM((128, F), jnp.float32),
            pltpu.VMEM((128, F), jnp.float32),
            pltpu.VMEM_SHARED((ACC_R, F), jnp.float32),
            pltpu.SemaphoreType.DMA,
            pltpu.SemaphoreType.DMA,
        ],
    )
    return deg, agg


def _deg(*args):
    return _sc_kernels()[0](*args)


def _agg(*args):
    return _sc_kernels()[1](*args)


def _mm_body(h_ref, w_ref, deg_ref, y_ref):
    dinv = lax.rsqrt(deg_ref[...][:, 0:1] + 1.0)
    y_ref[...] = (dinv * jnp.dot(h_ref[...], w_ref[...],
                                 preferred_element_type=jnp.float32))[None]


def _mm(h, w, deg16):
    k = h.shape[1]
    return pl.pallas_call(
        _mm_body,
        grid=(NCH, 10),
        in_specs=[
            pl.BlockSpec((1000, k), lambda c, r: (r, 0)),
            pl.BlockSpec((k, F), lambda c, r: (0, c)),
            pl.BlockSpec((1000, 16), lambda c, r: (r, 0)),
        ],
        out_specs=pl.BlockSpec((1, 1000, F), lambda c, r: (c, r, 0)),
        out_shape=jax.ShapeDtypeStruct((NCH, N, F), jnp.float32),
    )(h, w, deg16)


def _bn_body(s_ref, y_ref, deg_ref, b_ref, g_ref, bt_ref, h_ref, m_ref):
    s = s_ref[0, :N, :]
    y = y_ref[0]
    dinv = lax.rsqrt(deg_ref[...][:, 0:1] + 1.0)
    t = dinv * (s + y) + b_ref[0:1, :]
    mu = jnp.mean(t, axis=0, keepdims=True)
    ctr = t - mu
    var = jnp.mean(ctr * ctr, axis=0, keepdims=True)
    hh = jnp.maximum(ctr * lax.rsqrt(var + EPS) * g_ref[0:1, :] + bt_ref[0:1, :],
                     0.0)
    h_ref[...] = hh
    m_ref[...] = jnp.broadcast_to(jnp.mean(hh, axis=0, keepdims=True), (8, F))


def _bn(s, y, deg16, b8, g8, bt8):
    return pl.pallas_call(
        _bn_body,
        grid=(NCH,),
        in_specs=[
            pl.BlockSpec((1, ACC_R, F), lambda c: (c, 0, 0)),
            pl.BlockSpec((1, N, F), lambda c: (c, 0, 0)),
            pl.BlockSpec((N, 16), lambda c: (0, 0)),
            pl.BlockSpec((8, F), lambda c: (0, c)),
            pl.BlockSpec((8, F), lambda c: (0, c)),
            pl.BlockSpec((8, F), lambda c: (0, c)),
        ],
        out_specs=[
            pl.BlockSpec((N, F), lambda c: (0, c)),
            pl.BlockSpec((8, F), lambda c: (0, c)),
        ],
        out_shape=[
            jax.ShapeDtypeStruct((N, DH), jnp.float32),
            jax.ShapeDtypeStruct((8, DH), jnp.float32),
        ],
    )(s, y, deg16, b8, g8, bt8)


def kernel(x, edge_index, W0, b0, g0, bt0, W1, b1, g1, bt1,
           W2, b2, g2, bt2, W3, b3, g3, bt3):
    ei = edge_index.astype(jnp.int32)
    pad = EPAD - E
    srcp = jnp.concatenate([ei[0], jnp.zeros((pad,), jnp.int32)])
    dstp = jnp.concatenate([ei[1], jnp.full((pad,), N, jnp.int32)])
    dst16 = dstp.reshape(16, 80, 128)
    onesF = jnp.ones((128, F), jnp.float32)
    zerosF = jnp.zeros((128, F), jnp.float32)

    deg16 = _deg(dst16, onesF, zerosF)[:N, :16]

    h = x
    m = None
    for W, b, g, bt in ((W0, b0, g0, bt0), (W1, b1, g1, bt1),
                        (W2, b2, g2, bt2), (W3, b3, g3, bt3)):
        y = _mm(h, W, deg16)
        s = _agg(y, srcp, dst16, zerosF)
        b8 = jnp.broadcast_to(b.reshape(1, DH), (8, DH))
        g8 = jnp.broadcast_to(g.reshape(1, DH), (8, DH))
        bt8 = jnp.broadcast_to(bt.reshape(1, DH), (8, DH))
        h, m = _bn(s, y, deg16, b8, g8, bt8)
    return h, m[0:1]


# speedup vs baseline: 4.9816x; 1.0117x over previous
import functools

import jax
import jax.numpy as jnp
from jax import lax
from jax.experimental import pallas as pl
from jax.experimental.pallas import tpu as pltpu
from jax.experimental.pallas import tpu_sc as plsc

N = 10000
E = 160000
DH = 512
F = 128
NCH = DH // F
EPS = 1e-5
NTILE = 32
EPAD = NTILE * 40 * 128
ACC_R = 10240
SLAB = ACC_R // 16
HSLAB = SLAB // 2

def _deg_body(dst16_hbm, ones_hbm, zeros_hbm, out_hbm, dst_v, ones_v, zb, acc):
    cid = lax.axis_index("c")
    sid = lax.axis_index("s")
    pltpu.sync_copy(dst16_hbm.at[pl.ds(sid, 1)], dst_v)
    pltpu.sync_copy(ones_hbm, ones_v)
    pltpu.sync_copy(zeros_hbm, zb)
    r0 = pl.multiple_of(sid * SLAB, 8)
    for z in range(5):
        pltpu.sync_copy(zb, acc.at[pl.ds(r0 + z * 128, 128)])
    plsc.subcore_barrier()
    for j in range(80):
        pltpu.sync_copy(ones_v, acc.at[dst_v.at[0, j]], add=True)
    plsc.subcore_barrier()

    @pl.when(cid == 0)
    def _():
        for z in range(5):
            pltpu.sync_copy(acc.at[pl.ds(r0 + z * 128, 128)], zb)
            pltpu.sync_copy(zb, out_hbm.at[pl.ds(r0 + z * 128, 128)])


def _agg_body(y_hbm, src_hbm, dst_hbm, zeros_hbm, out_hbm,
              src_v, dst_v, gb0, gb1, acc, sem0, sem1, sem2, sem3):
    cid = lax.axis_index("c")
    sid = lax.axis_index("s")
    ebase = pl.multiple_of(sid * 10240, 8)
    pltpu.sync_copy(dst_hbm.at[pl.ds(sid, 1)], dst_v)
    r0 = pl.multiple_of(sid * SLAB, 8)
    bufs = (gb0, gb1)
    gsems = (sem0, sem1)
    ssems = (sem2, sem3)
    for k in range(NCH // 2):
        chunk = cid + 2 * k
        yv = y_hbm.at[chunk]
        pltpu.sync_copy(zeros_hbm, gb0)
        for z in range(5):
            pltpu.sync_copy(gb0, acc.at[pl.ds(r0 + z * 128, 128)])
        plsc.subcore_barrier()
        for half in range(2):
            pltpu.sync_copy(
                src_hbm.at[pl.ds(ebase + half * 5120, 5120)], src_v)
            g = pltpu.async_copy(yv.at[src_v.at[pl.ds(0, 128)]], gb0, sem0)
            s_prev = None
            for j in range(40):
                g.wait()
                s = pltpu.async_copy(bufs[j % 2],
                                     acc.at[dst_v.at[0, half * 40 + j]],
                                     ssems[j % 2], add=True)
                if s_prev is not None:
                    s_prev.wait()
                if j < 39:
                    g = pltpu.async_copy(
                        yv.at[src_v.at[pl.ds((j + 1) * 128, 128)]],
                        bufs[(j + 1) % 2], gsems[(j + 1) % 2])
                s_prev = s
            s_prev.wait()
        plsc.subcore_barrier()
        for z in range(5):
            b = bufs[z % 2]
            pltpu.sync_copy(acc.at[pl.ds(r0 + z * 128, 128)], b)
            pltpu.sync_copy(b, out_hbm.at[chunk, pl.ds(r0 + z * 128, 128)])
        plsc.subcore_barrier()


@functools.cache
def _sc_kernels():
    mesh = plsc.VectorSubcoreMesh(core_axis_name="c", subcore_axis_name="s")
    deg = pl.kernel(
        _deg_body,
        out_type=jax.ShapeDtypeStruct((ACC_R, F), jnp.float32),
        mesh=mesh,
        scratch_types=[
            pltpu.VMEM((1, 80, 128), jnp.int32),
            pltpu.VMEM((128, F), jnp.float32),
            pltpu.VMEM((128, F), jnp.float32),
            pltpu.VMEM_SHARED((ACC_R, F), jnp.float32),
        ],
    )
    agg = pl.kernel(
        _agg_body,
        out_type=jax.ShapeDtypeStruct((NCH, ACC_R, F), jnp.float32),
        mesh=mesh,
        scratch_types=[
            pltpu.VMEM((5120,), jnp.int32),
            pltpu.VMEM((1, 80, 128), jnp.int32),
            pltpu.VMEM((128, F), jnp.float32),
            pltpu.VMEM((128, F), jnp.float32),
            pltpu.VMEM_SHARED((ACC_R, F), jnp.float32),
            pltpu.SemaphoreType.DMA,
            pltpu.SemaphoreType.DMA,
            pltpu.SemaphoreType.DMA,
            pltpu.SemaphoreType.DMA,
        ],
    )
    return deg, agg


def _deg(*args):
    return _sc_kernels()[0](*args)


def _agg(*args):
    return _sc_kernels()[1](*args)


def _mm_body(h_ref, w_ref, deg_ref, y_ref):
    dinv = lax.rsqrt(deg_ref[...][:, 0:1] + 1.0)
    y_ref[...] = (dinv * jnp.dot(h_ref[...], w_ref[...],
                                 preferred_element_type=jnp.float32))[None]


def _mm(h, w, deg16):
    k = h.shape[1]
    return pl.pallas_call(
        _mm_body,
        grid=(NCH, 10),
        in_specs=[
            pl.BlockSpec((1000, k), lambda c, r: (r, 0)),
            pl.BlockSpec((k, F), lambda c, r: (0, c)),
            pl.BlockSpec((1000, 16), lambda c, r: (r, 0)),
        ],
        out_specs=pl.BlockSpec((1, 1000, F), lambda c, r: (c, r, 0)),
        out_shape=jax.ShapeDtypeStruct((NCH, N, F), jnp.float32),
    )(h, w, deg16)


def _bn_body(s_ref, y_ref, deg_ref, b_ref, g_ref, bt_ref, h_ref, m_ref):
    s = s_ref[0, :N, :]
    y = y_ref[0]
    dinv = lax.rsqrt(deg_ref[...][:, 0:1] + 1.0)
    t = dinv * (s + y) + b_ref[0:1, :]
    mu = jnp.mean(t, axis=0, keepdims=True)
    ctr = t - mu
    var = jnp.mean(ctr * ctr, axis=0, keepdims=True)
    hh = jnp.maximum(ctr * lax.rsqrt(var + EPS) * g_ref[0:1, :] + bt_ref[0:1, :],
                     0.0)
    h_ref[...] = hh
    m_ref[...] = jnp.broadcast_to(jnp.mean(hh, axis=0, keepdims=True), (8, F))


def _bn(s, y, deg16, b8, g8, bt8):
    return pl.pallas_call(
        _bn_body,
        grid=(NCH,),
        in_specs=[
            pl.BlockSpec((1, ACC_R, F), lambda c: (c, 0, 0)),
            pl.BlockSpec((1, N, F), lambda c: (c, 0, 0)),
            pl.BlockSpec((N, 16), lambda c: (0, 0)),
            pl.BlockSpec((8, F), lambda c: (0, c)),
            pl.BlockSpec((8, F), lambda c: (0, c)),
            pl.BlockSpec((8, F), lambda c: (0, c)),
        ],
        out_specs=[
            pl.BlockSpec((N, F), lambda c: (0, c)),
            pl.BlockSpec((8, F), lambda c: (0, c)),
        ],
        out_shape=[
            jax.ShapeDtypeStruct((N, DH), jnp.float32),
            jax.ShapeDtypeStruct((8, DH), jnp.float32),
        ],
    )(s, y, deg16, b8, g8, bt8)


def kernel(x, edge_index, W0, b0, g0, bt0, W1, b1, g1, bt1,
           W2, b2, g2, bt2, W3, b3, g3, bt3):
    ei = edge_index.astype(jnp.int32)
    pad = EPAD - E
    srcp = jnp.concatenate([ei[0], jnp.zeros((pad,), jnp.int32)])
    dstp = jnp.concatenate([ei[1], jnp.full((pad,), N, jnp.int32)])
    dst16 = dstp.reshape(16, 80, 128)
    onesF = jnp.ones((128, F), jnp.float32)
    zerosF = jnp.zeros((128, F), jnp.float32)

    deg16 = _deg(dst16, onesF, zerosF)[:N, :16]

    h = x
    m = None
    for W, b, g, bt in ((W0, b0, g0, bt0), (W1, b1, g1, bt1),
                        (W2, b2, g2, bt2), (W3, b3, g3, bt3)):
        y = _mm(h, W, deg16)
        s = _agg(y, srcp, dst16, zerosF)
        b8 = jnp.broadcast_to(b.reshape(1, DH), (8, DH))
        g8 = jnp.broadcast_to(g.reshape(1, DH), (8, DH))
        bt8 = jnp.broadcast_to(bt.reshape(1, DH), (8, DH))
        h, m = _bn(s, y, deg16, b8, g8, bt8)
    return h, m[0:1]


# speedup vs baseline: 5.2461x; 1.0531x over previous
import functools

import jax
import jax.numpy as jnp
from jax import lax
from jax.experimental import pallas as pl
from jax.experimental.pallas import tpu as pltpu
from jax.experimental.pallas import tpu_sc as plsc

N = 10000
E = 160000
DH = 512
F = 128
NCH = DH // F
EPS = 1e-5
NTILE = 32
EPAD = NTILE * 40 * 128
ACC_R = 10112
SLAB = ACC_R // 16
HSLAB = SLAB // 2

def _deg_body(dst16_hbm, ones_hbm, zeros_hbm, out_hbm, dst_v, ones_v, zb, acc):
    cid = lax.axis_index("c")
    sid = lax.axis_index("s")
    pltpu.sync_copy(dst16_hbm.at[pl.ds(sid, 1)], dst_v)
    pltpu.sync_copy(ones_hbm, ones_v)
    pltpu.sync_copy(zeros_hbm, zb)
    r0 = pl.multiple_of(sid * SLAB, 8)
    for z in range(8):
        w = 80 if z < 7 else 72
        pltpu.sync_copy(zb.at[pl.ds(0, w)], acc.at[pl.ds(r0 + z * 80, w)])
    plsc.subcore_barrier()
    for j in range(80):
        pltpu.sync_copy(ones_v, acc.at[dst_v.at[0, j]], add=True)
    plsc.subcore_barrier()

    @pl.when(cid == 0)
    def _():
        for z in range(8):
            w = 80 if z < 7 else 72
            pltpu.sync_copy(acc.at[pl.ds(r0 + z * 80, w)], zb.at[pl.ds(0, w)])
            pltpu.sync_copy(zb.at[pl.ds(0, w)], out_hbm.at[pl.ds(r0 + z * 80, w)])


def _agg_body(y_hbm, src_hbm, dst_hbm, zeros_hbm, out_hbm,
              src_v, dst_v, gb0, gb1, gb2, acc,
              sem0, sem1, sem2, sem3, sem4, sem5):
    cid = lax.axis_index("c")
    sid = lax.axis_index("s")
    ebase = pl.multiple_of(sid * 10240, 8)
    pltpu.sync_copy(dst_hbm.at[pl.ds(sid, 1)], dst_v)
    r0 = pl.multiple_of(sid * SLAB, 8)
    bufs = (gb0, gb1, gb2)
    gsems = (sem0, sem1, sem2)
    ssems = (sem3, sem4, sem5)
    NB = 32
    for k in range(NCH // 2):
        chunk = cid + 2 * k
        yv = y_hbm.at[chunk]
        pltpu.sync_copy(zeros_hbm, gb0)
        for z in range(8):
            w = 80 if z < 7 else 72
            pltpu.sync_copy(gb0.at[pl.ds(0, w)], acc.at[pl.ds(r0 + z * 80, w)])
        plsc.subcore_barrier()
        for q in range(4):
            pltpu.sync_copy(
                src_hbm.at[pl.ds(ebase + q * 2560, 2560)], src_v)

            def gat(j):
                return pltpu.async_copy(
                    yv.at[src_v.at[pl.ds(j * 80, 80)]],
                    bufs[j % 3], gsems[j % 3])

            g = {0: gat(0), 1: gat(1)}
            s = {}
            for j in range(NB):
                g.pop(j).wait()
                s[j] = pltpu.async_copy(bufs[j % 3],
                                        acc.at[dst_v.at[0, q * NB + j]],
                                        ssems[j % 3], add=True)
                if j > 0:
                    s.pop(j - 1).wait()
                if j + 2 < NB:
                    g[j + 2] = gat(j + 2)
            s.pop(NB - 1).wait()
        plsc.subcore_barrier()
        for z in range(8):
            w = 80 if z < 7 else 72
            b = bufs[z % 3]
            pltpu.sync_copy(acc.at[pl.ds(r0 + z * 80, w)], b.at[pl.ds(0, w)])
            pltpu.sync_copy(b.at[pl.ds(0, w)],
                            out_hbm.at[chunk, pl.ds(r0 + z * 80, w)])
        plsc.subcore_barrier()


@functools.cache
def _sc_kernels():
    mesh = plsc.VectorSubcoreMesh(core_axis_name="c", subcore_axis_name="s")
    deg = pl.kernel(
        _deg_body,
        out_type=jax.ShapeDtypeStruct((ACC_R, F), jnp.float32),
        mesh=mesh,
        scratch_types=[
            pltpu.VMEM((1, 80, 128), jnp.int32),
            pltpu.VMEM((128, F), jnp.float32),
            pltpu.VMEM((80, F), jnp.float32),
            pltpu.VMEM_SHARED((ACC_R, F), jnp.float32),
        ],
    )
    agg = pl.kernel(
        _agg_body,
        out_type=jax.ShapeDtypeStruct((NCH, ACC_R, F), jnp.float32),
        mesh=mesh,
        scratch_types=[
            pltpu.VMEM((2560,), jnp.int32),
            pltpu.VMEM((1, 128, 80), jnp.int32),
            pltpu.VMEM((80, F), jnp.float32),
            pltpu.VMEM((80, F), jnp.float32),
            pltpu.VMEM((80, F), jnp.float32),
            pltpu.VMEM_SHARED((ACC_R, F), jnp.float32),
            pltpu.SemaphoreType.DMA,
            pltpu.SemaphoreType.DMA,
            pltpu.SemaphoreType.DMA,
            pltpu.SemaphoreType.DMA,
            pltpu.SemaphoreType.DMA,
            pltpu.SemaphoreType.DMA,
        ],
    )
    return deg, agg


def _deg(*args):
    return _sc_kernels()[0](*args)


def _agg(*args):
    return _sc_kernels()[1](*args)


def _mm_body(h_ref, w_ref, deg_ref, y_ref):
    dinv = lax.rsqrt(deg_ref[...][:, 0:1] + 1.0)
    y_ref[...] = (dinv * jnp.dot(h_ref[...], w_ref[...],
                                 preferred_element_type=jnp.float32))[None]


def _mm(h, w, deg16):
    k = h.shape[1]
    return pl.pallas_call(
        _mm_body,
        grid=(NCH, 10),
        in_specs=[
            pl.BlockSpec((1000, k), lambda c, r: (r, 0)),
            pl.BlockSpec((k, F), lambda c, r: (0, c)),
            pl.BlockSpec((1000, 16), lambda c, r: (r, 0)),
        ],
        out_specs=pl.BlockSpec((1, 1000, F), lambda c, r: (c, r, 0)),
        out_shape=jax.ShapeDtypeStruct((NCH, N, F), jnp.float32),
    )(h, w, deg16)


def _bn_body(s_ref, y_ref, deg_ref, b_ref, g_ref, bt_ref, h_ref, m_ref):
    s = s_ref[0, :N, :]
    y = y_ref[0]
    dinv = lax.rsqrt(deg_ref[...][:, 0:1] + 1.0)
    t = dinv * (s + y) + b_ref[0:1, :]
    mu = jnp.mean(t, axis=0, keepdims=True)
    ctr = t - mu
    var = jnp.mean(ctr * ctr, axis=0, keepdims=True)
    hh = jnp.maximum(ctr * lax.rsqrt(var + EPS) * g_ref[0:1, :] + bt_ref[0:1, :],
                     0.0)
    h_ref[...] = hh
    m_ref[...] = jnp.broadcast_to(jnp.mean(hh, axis=0, keepdims=True), (8, F))


def _bn(s, y, deg16, b8, g8, bt8):
    return pl.pallas_call(
        _bn_body,
        grid=(NCH,),
        in_specs=[
            pl.BlockSpec((1, ACC_R, F), lambda c: (c, 0, 0)),
            pl.BlockSpec((1, N, F), lambda c: (c, 0, 0)),
            pl.BlockSpec((N, 16), lambda c: (0, 0)),
            pl.BlockSpec((8, F), lambda c: (0, c)),
            pl.BlockSpec((8, F), lambda c: (0, c)),
            pl.BlockSpec((8, F), lambda c: (0, c)),
        ],
        out_specs=[
            pl.BlockSpec((N, F), lambda c: (0, c)),
            pl.BlockSpec((8, F), lambda c: (0, c)),
        ],
        out_shape=[
            jax.ShapeDtypeStruct((N, DH), jnp.float32),
            jax.ShapeDtypeStruct((8, DH), jnp.float32),
        ],
    )(s, y, deg16, b8, g8, bt8)


def kernel(x, edge_index, W0, b0, g0, bt0, W1, b1, g1, bt1,
           W2, b2, g2, bt2, W3, b3, g3, bt3):
    ei = edge_index.astype(jnp.int32)
    pad = EPAD - E
    srcp = jnp.concatenate([ei[0], jnp.zeros((pad,), jnp.int32)])
    dstp = jnp.concatenate([ei[1], jnp.full((pad,), N, jnp.int32)])
    dst16 = dstp.reshape(16, 80, 128)
    dst_a = dstp.reshape(16, 128, 80)
    onesF = jnp.ones((128, F), jnp.float32)
    zerosF = jnp.zeros((80, F), jnp.float32)

    deg16 = _deg(dst16, onesF, zerosF)[:N, :16]

    h = x
    m = None
    for W, b, g, bt in ((W0, b0, g0, bt0), (W1, b1, g1, bt1),
                        (W2, b2, g2, bt2), (W3, b3, g3, bt3)):
        y = _mm(h, W, deg16)
        s = _agg(y, srcp, dst_a, zerosF)
        b8 = jnp.broadcast_to(b.reshape(1, DH), (8, DH))
        g8 = jnp.broadcast_to(g.reshape(1, DH), (8, DH))
        bt8 = jnp.broadcast_to(bt.reshape(1, DH), (8, DH))
        h, m = _bn(s, y, deg16, b8, g8, bt8)
    return h, m[0:1]


# speedup vs baseline: 5.6674x; 1.0803x over previous
import functools

import jax
import jax.numpy as jnp
from jax import lax
from jax.experimental import pallas as pl
from jax.experimental.pallas import tpu as pltpu
from jax.experimental.pallas import tpu_sc as plsc

N = 10000
E = 160000
DH = 512
F = 128
NCH = DH // F
EPS = 1e-5
NTILE = 32
EPAD = NTILE * 40 * 128
ACC_R = 10112
SLAB = ACC_R // 16
HSLAB = SLAB // 2

def _deg_body(dst16_hbm, ones_hbm, zeros_hbm, out_hbm, dst_v, ones_v, zb, acc):
    cid = lax.axis_index("c")
    sid = lax.axis_index("s")
    pltpu.sync_copy(dst16_hbm.at[pl.ds(sid, 1)], dst_v)
    pltpu.sync_copy(ones_hbm, ones_v)
    pltpu.sync_copy(zeros_hbm, zb)
    r0 = pl.multiple_of(sid * SLAB, 8)
    for z in range(8):
        w = 80 if z < 7 else 72
        pltpu.sync_copy(zb.at[pl.ds(0, w)], acc.at[pl.ds(r0 + z * 80, w)])
    plsc.subcore_barrier()
    for j in range(80):
        pltpu.sync_copy(ones_v, acc.at[dst_v.at[0, j]], add=True)
    plsc.subcore_barrier()

    @pl.when(cid == 0)
    def _():
        for z in range(8):
            w = 80 if z < 7 else 72
            pltpu.sync_copy(acc.at[pl.ds(r0 + z * 80, w)], zb.at[pl.ds(0, w)])
            pltpu.sync_copy(zb.at[pl.ds(0, w)], out_hbm.at[pl.ds(r0 + z * 80, w)])


def _agg_body(nch, y_hbm, src_hbm, dst_hbm, zeros_hbm, out_hbm,
              src_v, dst_v, gb0, gb1, gb2, acc,
              sem0, sem1, sem2, sem3, sem4, sem5):
    cid = lax.axis_index("c")
    sid = lax.axis_index("s")
    ebase = pl.multiple_of(sid * 10240, 8)
    pltpu.sync_copy(dst_hbm.at[pl.ds(sid, 1)], dst_v)
    r0 = pl.multiple_of(sid * SLAB, 8)
    bufs = (gb0, gb1, gb2)
    gsems = (sem0, sem1, sem2)
    ssems = (sem3, sem4, sem5)
    NB = 32
    for k in range(nch // 2):
        chunk = cid + 2 * k
        yv = y_hbm.at[chunk]
        pltpu.sync_copy(zeros_hbm, gb0)
        for z in range(8):
            w = 80 if z < 7 else 72
            pltpu.sync_copy(gb0.at[pl.ds(0, w)], acc.at[pl.ds(r0 + z * 80, w)])
        plsc.subcore_barrier()
        for q in range(4):
            pltpu.sync_copy(
                src_hbm.at[pl.ds(ebase + q * 2560, 2560)], src_v)

            def gat(j):
                return pltpu.async_copy(
                    yv.at[src_v.at[pl.ds(j * 80, 80)]],
                    bufs[j % 3], gsems[j % 3])

            g = {0: gat(0), 1: gat(1)}
            s = {}
            for j in range(NB):
                g.pop(j).wait()
                s[j] = pltpu.async_copy(bufs[j % 3],
                                        acc.at[dst_v.at[0, q * NB + j]],
                                        ssems[j % 3], add=True)
                if j > 0:
                    s.pop(j - 1).wait()
                if j + 2 < NB:
                    g[j + 2] = gat(j + 2)
            s.pop(NB - 1).wait()
        plsc.subcore_barrier()
        for z in range(8):
            w = 80 if z < 7 else 72
            b = bufs[z % 3]
            pltpu.sync_copy(acc.at[pl.ds(r0 + z * 80, w)], b.at[pl.ds(0, w)])
            pltpu.sync_copy(b.at[pl.ds(0, w)],
                            out_hbm.at[chunk, pl.ds(r0 + z * 80, w)])
        plsc.subcore_barrier()


@functools.cache
def _sc_kernels():
    mesh = plsc.VectorSubcoreMesh(core_axis_name="c", subcore_axis_name="s")
    deg = pl.kernel(
        _deg_body,
        out_type=jax.ShapeDtypeStruct((ACC_R, F), jnp.float32),
        mesh=mesh,
        scratch_types=[
            pltpu.VMEM((1, 80, 128), jnp.int32),
            pltpu.VMEM((128, F), jnp.float32),
            pltpu.VMEM((80, F), jnp.float32),
            pltpu.VMEM_SHARED((ACC_R, F), jnp.float32),
        ],
    )
    def mkagg(nch):
        return pl.kernel(
        functools.partial(_agg_body, nch),
        out_type=jax.ShapeDtypeStruct((nch, ACC_R, F), jnp.float32),
        mesh=mesh,
        scratch_types=[
            pltpu.VMEM((2560,), jnp.int32),
            pltpu.VMEM((1, 128, 80), jnp.int32),
            pltpu.VMEM((80, F), jnp.float32),
            pltpu.VMEM((80, F), jnp.float32),
            pltpu.VMEM((80, F), jnp.float32),
            pltpu.VMEM_SHARED((ACC_R, F), jnp.float32),
            pltpu.SemaphoreType.DMA,
            pltpu.SemaphoreType.DMA,
            pltpu.SemaphoreType.DMA,
            pltpu.SemaphoreType.DMA,
            pltpu.SemaphoreType.DMA,
            pltpu.SemaphoreType.DMA,
        ],
        )
    return deg, mkagg(NCH), mkagg(2)


def _deg(*args):
    return _sc_kernels()[0](*args)


def _agg(*args):
    return _sc_kernels()[1](*args)


def _agg2(*args):
    return _sc_kernels()[2](*args)


def _mm_body(h_ref, w_ref, deg_ref, y_ref):
    dinv = lax.rsqrt(deg_ref[...][:, 0:1] + 1.0)
    y_ref[...] = (dinv * jnp.dot(h_ref[...], w_ref[...],
                                 preferred_element_type=jnp.float32))[None]


def _mm(h, w, deg16):
    k = h.shape[1]
    return pl.pallas_call(
        _mm_body,
        grid=(NCH, 10),
        in_specs=[
            pl.BlockSpec((1000, k), lambda c, r: (r, 0)),
            pl.BlockSpec((k, F), lambda c, r: (0, c)),
            pl.BlockSpec((1000, 16), lambda c, r: (r, 0)),
        ],
        out_specs=pl.BlockSpec((1, 1000, F), lambda c, r: (c, r, 0)),
        out_shape=jax.ShapeDtypeStruct((NCH, N, F), jnp.float32),
    )(h, w, deg16)


def _scale_body(x_ref, deg_ref, y_ref):
    dinv = lax.rsqrt(deg_ref[...][:, 0:1] + 1.0)
    y_ref[...] = (dinv * x_ref[...])[None]


def _scale(x, deg16):
    return pl.pallas_call(
        _scale_body,
        grid=(2, 10),
        in_specs=[
            pl.BlockSpec((1000, F), lambda c, r: (r, c)),
            pl.BlockSpec((1000, 16), lambda c, r: (r, 0)),
        ],
        out_specs=pl.BlockSpec((1, 1000, F), lambda c, r: (c, r, 0)),
        out_shape=jax.ShapeDtypeStruct((2, N, F), jnp.float32),
    )(x, deg16)


def _mm0_body(s_ref, y_ref, deg_ref, w_ref, o_ref):
    dinv = lax.rsqrt(deg_ref[...][:, 0:1] + 1.0)
    s2 = jnp.concatenate([s_ref[0], s_ref[1]], axis=1)
    y2 = jnp.concatenate([y_ref[0], y_ref[1]], axis=1)
    a = dinv * (s2 + y2)
    o_ref[...] = jnp.dot(a, w_ref[...], preferred_element_type=jnp.float32)[None]


def _mm0(s, y, deg16, w):
    return pl.pallas_call(
        _mm0_body,
        grid=(NCH, 10),
        in_specs=[
            pl.BlockSpec((2, 1000, F), lambda c, r: (0, r, 0)),
            pl.BlockSpec((2, 1000, F), lambda c, r: (0, r, 0)),
            pl.BlockSpec((1000, 16), lambda c, r: (r, 0)),
            pl.BlockSpec((2 * F, F), lambda c, r: (0, c)),
        ],
        out_specs=pl.BlockSpec((1, 1000, F), lambda c, r: (c, r, 0)),
        out_shape=jax.ShapeDtypeStruct((NCH, N, F), jnp.float32),
    )(s, y, deg16, w)


def _bn_body(s_ref, y_ref, deg_ref, b_ref, g_ref, bt_ref, h_ref, m_ref):
    s = s_ref[0, :N, :]
    y = y_ref[0]
    dinv = lax.rsqrt(deg_ref[...][:, 0:1] + 1.0)
    t = dinv * (s + y) + b_ref[0:1, :]
    mu = jnp.mean(t, axis=0, keepdims=True)
    ctr = t - mu
    var = jnp.mean(ctr * ctr, axis=0, keepdims=True)
    hh = jnp.maximum(ctr * lax.rsqrt(var + EPS) * g_ref[0:1, :] + bt_ref[0:1, :],
                     0.0)
    h_ref[...] = hh
    m_ref[...] = jnp.broadcast_to(jnp.mean(hh, axis=0, keepdims=True), (8, F))


def _bn0_body(y_ref, b_ref, g_ref, bt_ref, h_ref, m_ref):
    t = y_ref[0] + b_ref[0:1, :]
    mu = jnp.mean(t, axis=0, keepdims=True)
    ctr = t - mu
    var = jnp.mean(ctr * ctr, axis=0, keepdims=True)
    hh = jnp.maximum(ctr * lax.rsqrt(var + EPS) * g_ref[0:1, :] + bt_ref[0:1, :],
                     0.0)
    h_ref[...] = hh
    m_ref[...] = jnp.broadcast_to(jnp.mean(hh, axis=0, keepdims=True), (8, F))


def _bn0(y, b8, g8, bt8):
    return pl.pallas_call(
        _bn0_body,
        grid=(NCH,),
        in_specs=[
            pl.BlockSpec((1, N, F), lambda c: (c, 0, 0)),
            pl.BlockSpec((8, F), lambda c: (0, c)),
            pl.BlockSpec((8, F), lambda c: (0, c)),
            pl.BlockSpec((8, F), lambda c: (0, c)),
        ],
        out_specs=[
            pl.BlockSpec((N, F), lambda c: (0, c)),
            pl.BlockSpec((8, F), lambda c: (0, c)),
        ],
        out_shape=[
            jax.ShapeDtypeStruct((N, DH), jnp.float32),
            jax.ShapeDtypeStruct((8, DH), jnp.float32),
        ],
    )(y, b8, g8, bt8)


def _bn(s, y, deg16, b8, g8, bt8):
    return pl.pallas_call(
        _bn_body,
        grid=(NCH,),
        in_specs=[
            pl.BlockSpec((1, ACC_R, F), lambda c: (c, 0, 0)),
            pl.BlockSpec((1, N, F), lambda c: (c, 0, 0)),
            pl.BlockSpec((N, 16), lambda c: (0, 0)),
            pl.BlockSpec((8, F), lambda c: (0, c)),
            pl.BlockSpec((8, F), lambda c: (0, c)),
            pl.BlockSpec((8, F), lambda c: (0, c)),
        ],
        out_specs=[
            pl.BlockSpec((N, F), lambda c: (0, c)),
            pl.BlockSpec((8, F), lambda c: (0, c)),
        ],
        out_shape=[
            jax.ShapeDtypeStruct((N, DH), jnp.float32),
            jax.ShapeDtypeStruct((8, DH), jnp.float32),
        ],
    )(s, y, deg16, b8, g8, bt8)


def kernel(x, edge_index, W0, b0, g0, bt0, W1, b1, g1, bt1,
           W2, b2, g2, bt2, W3, b3, g3, bt3):
    ei = edge_index.astype(jnp.int32)
    pad = EPAD - E
    srcp = jnp.concatenate([ei[0], jnp.zeros((pad,), jnp.int32)])
    dstp = jnp.concatenate([ei[1], jnp.full((pad,), N, jnp.int32)])
    dst16 = dstp.reshape(16, 80, 128)
    dst_a = dstp.reshape(16, 128, 80)
    onesF = jnp.ones((128, F), jnp.float32)
    zerosF = jnp.zeros((80, F), jnp.float32)

    deg16 = _deg(dst16, onesF, zerosF)[:N, :16]

    def p8(v):
        return jnp.broadcast_to(v.reshape(1, DH), (8, DH))

    y0 = _scale(x, deg16)
    s0 = _agg2(y0, srcp, dst_a, zerosF)
    xw0 = _mm0(s0, y0, deg16, W0)
    h, m = _bn0(xw0, p8(b0), p8(g0), p8(bt0))

    for W, b, g, bt in ((W1, b1, g1, bt1), (W2, b2, g2, bt2),
                        (W3, b3, g3, bt3)):
        y = _mm(h, W, deg16)
        s = _agg(y, srcp, dst_a, zerosF)
        h, m = _bn(s, y, deg16, p8(b), p8(g), p8(bt))
    return h, m[0:1]


# speedup vs baseline: 5.6698x; 1.0004x over previous
import functools

import jax
import jax.numpy as jnp
from jax import lax
from jax.experimental import pallas as pl
from jax.experimental.pallas import tpu as pltpu
from jax.experimental.pallas import tpu_sc as plsc

N = 10000
E = 160000
DH = 512
F = 128
NCH = DH // F
EPS = 1e-5
EPAD = 16 * 80 * 128
ACC_R = 10112
SLAB = ACC_R // 16

def _deg_body(dst16_hbm, ones_hbm, zeros_hbm, out_hbm, dst_v, ones_v, zb, acc):
    cid = lax.axis_index("c")
    sid = lax.axis_index("s")
    pltpu.sync_copy(dst16_hbm.at[pl.ds(sid, 1)], dst_v)
    pltpu.sync_copy(ones_hbm, ones_v)
    pltpu.sync_copy(zeros_hbm, zb)
    r0 = pl.multiple_of(sid * SLAB, 8)
    for z in range(8):
        w = 80 if z < 7 else 72
        pltpu.sync_copy(zb.at[pl.ds(0, w)], acc.at[pl.ds(r0 + z * 80, w)])
    plsc.subcore_barrier()
    for j in range(80):
        pltpu.sync_copy(ones_v, acc.at[dst_v.at[0, j]], add=True)
    plsc.subcore_barrier()

    @pl.when(cid == 0)
    def _():
        for z in range(8):
            w = 80 if z < 7 else 72
            pltpu.sync_copy(acc.at[pl.ds(r0 + z * 80, w)], zb.at[pl.ds(0, w)])
            pltpu.sync_copy(zb.at[pl.ds(0, w)], out_hbm.at[pl.ds(r0 + z * 80, w)])


def _agg_body(nch, y_hbm, src_hbm, dst_hbm, zeros_hbm, out_hbm,
              src_v, dst_v, gb0, gb1, gb2, acc,
              sem0, sem1, sem2, sem3, sem4, sem5):
    cid = lax.axis_index("c")
    sid = lax.axis_index("s")
    ebase = pl.multiple_of(sid * 10240, 8)
    pltpu.sync_copy(dst_hbm.at[pl.ds(sid, 1)], dst_v)
    r0 = pl.multiple_of(sid * SLAB, 8)
    bufs = (gb0, gb1, gb2)
    gsems = (sem0, sem1, sem2)
    ssems = (sem3, sem4, sem5)
    NB = 32
    for k in range(nch // 2):
        chunk = cid + 2 * k
        yv = y_hbm.at[chunk]
        pltpu.sync_copy(zeros_hbm, gb0)
        for z in range(8):
            w = 80 if z < 7 else 72
            pltpu.sync_copy(gb0.at[pl.ds(0, w)], acc.at[pl.ds(r0 + z * 80, w)])
        plsc.subcore_barrier()
        for q in range(4):
            pltpu.sync_copy(
                src_hbm.at[pl.ds(ebase + q * 2560, 2560)], src_v)

            def gat(j):
                return pltpu.async_copy(
                    yv.at[src_v.at[pl.ds(j * 80, 80)]],
                    bufs[j % 3], gsems[j % 3])

            g = {0: gat(0), 1: gat(1)}
            s = {}
            for j in range(NB):
                g.pop(j).wait()
                s[j] = pltpu.async_copy(bufs[j % 3],
                                        acc.at[dst_v.at[0, q * NB + j]],
                                        ssems[j % 3], add=True)
                if j > 0:
                    s.pop(j - 1).wait()
                if j + 2 < NB:
                    g[j + 2] = gat(j + 2)
            s.pop(NB - 1).wait()
        plsc.subcore_barrier()
        for z in range(8):
            w = 80 if z < 7 else 72
            b = bufs[z % 3]
            pltpu.sync_copy(acc.at[pl.ds(r0 + z * 80, w)], b.at[pl.ds(0, w)])
            pltpu.sync_copy(b.at[pl.ds(0, w)],
                            out_hbm.at[chunk, pl.ds(r0 + z * 80, w)])
        plsc.subcore_barrier()


@functools.cache
def _sc_kernels():
    mesh = plsc.VectorSubcoreMesh(core_axis_name="c", subcore_axis_name="s")
    deg = pl.kernel(
        _deg_body,
        out_type=jax.ShapeDtypeStruct((ACC_R, F), jnp.float32),
        mesh=mesh,
        scratch_types=[
            pltpu.VMEM((1, 80, 128), jnp.int32),
            pltpu.VMEM((128, F), jnp.float32),
            pltpu.VMEM((80, F), jnp.float32),
            pltpu.VMEM_SHARED((ACC_R, F), jnp.float32),
        ],
    )
    def mkagg(nch):
        return pl.kernel(
        functools.partial(_agg_body, nch),
        out_type=jax.ShapeDtypeStruct((nch, ACC_R, F), jnp.float32),
        mesh=mesh,
        scratch_types=[
            pltpu.VMEM((2560,), jnp.int32),
            pltpu.VMEM((1, 128, 80), jnp.int32),
            pltpu.VMEM((80, F), jnp.float32),
            pltpu.VMEM((80, F), jnp.float32),
            pltpu.VMEM((80, F), jnp.float32),
            pltpu.VMEM_SHARED((ACC_R, F), jnp.float32),
            pltpu.SemaphoreType.DMA,
            pltpu.SemaphoreType.DMA,
            pltpu.SemaphoreType.DMA,
            pltpu.SemaphoreType.DMA,
            pltpu.SemaphoreType.DMA,
            pltpu.SemaphoreType.DMA,
        ],
        )
    return deg, mkagg(NCH), mkagg(2)


def _deg(*args):
    return _sc_kernels()[0](*args)


def _agg(*args):
    return _sc_kernels()[1](*args)


def _agg2(*args):
    return _sc_kernels()[2](*args)


def _mm_body(h_ref, w_ref, deg_ref, y_ref):
    dinv = lax.rsqrt(deg_ref[...][:, 0:1] + 1.0)
    y_ref[...] = (dinv * jnp.dot(h_ref[...], w_ref[...],
                                 preferred_element_type=jnp.float32))[None]


def _mm(h, w, deg16):
    k = h.shape[1]
    return pl.pallas_call(
        _mm_body,
        grid=(NCH, 10),
        in_specs=[
            pl.BlockSpec((1000, k), lambda c, r: (r, 0)),
            pl.BlockSpec((k, F), lambda c, r: (0, c)),
            pl.BlockSpec((1000, 16), lambda c, r: (r, 0)),
        ],
        out_specs=pl.BlockSpec((1, 1000, F), lambda c, r: (c, r, 0)),
        out_shape=jax.ShapeDtypeStruct((NCH, N, F), jnp.float32),
    )(h, w, deg16)


def _scale_body(x_ref, deg_ref, y_ref):
    dinv = lax.rsqrt(deg_ref[...][:, 0:1] + 1.0)
    y_ref[...] = (dinv * x_ref[...])[None]


def _scale(x, deg16):
    return pl.pallas_call(
        _scale_body,
        grid=(2, 10),
        in_specs=[
            pl.BlockSpec((1000, F), lambda c, r: (r, c)),
            pl.BlockSpec((1000, 16), lambda c, r: (r, 0)),
        ],
        out_specs=pl.BlockSpec((1, 1000, F), lambda c, r: (c, r, 0)),
        out_shape=jax.ShapeDtypeStruct((2, N, F), jnp.float32),
    )(x, deg16)


def _mm0_body(s_ref, y_ref, deg_ref, w_ref, o_ref):
    dinv = lax.rsqrt(deg_ref[...][:, 0:1] + 1.0)
    s2 = jnp.concatenate([s_ref[0], s_ref[1]], axis=1)
    y2 = jnp.concatenate([y_ref[0], y_ref[1]], axis=1)
    a = dinv * (s2 + y2)
    o_ref[...] = jnp.dot(a, w_ref[...], preferred_element_type=jnp.float32)[None]


def _mm0(s, y, deg16, w):
    return pl.pallas_call(
        _mm0_body,
        grid=(NCH, 10),
        in_specs=[
            pl.BlockSpec((2, 1000, F), lambda c, r: (0, r, 0)),
            pl.BlockSpec((2, 1000, F), lambda c, r: (0, r, 0)),
            pl.BlockSpec((1000, 16), lambda c, r: (r, 0)),
            pl.BlockSpec((2 * F, F), lambda c, r: (0, c)),
        ],
        out_specs=pl.BlockSpec((1, 1000, F), lambda c, r: (c, r, 0)),
        out_shape=jax.ShapeDtypeStruct((NCH, N, F), jnp.float32),
    )(s, y, deg16, w)


def _bn_body(s_ref, y_ref, deg_ref, b_ref, g_ref, bt_ref, h_ref, m_ref):
    s = s_ref[0, :N, :]
    y = y_ref[0]
    dinv = lax.rsqrt(deg_ref[...][:, 0:1] + 1.0)
    t = dinv * (s + y) + b_ref[0:1, :]
    mu = jnp.mean(t, axis=0, keepdims=True)
    ctr = t - mu
    var = jnp.mean(ctr * ctr, axis=0, keepdims=True)
    hh = jnp.maximum(ctr * lax.rsqrt(var + EPS) * g_ref[0:1, :] + bt_ref[0:1, :],
                     0.0)
    h_ref[...] = hh
    m_ref[...] = jnp.broadcast_to(jnp.mean(hh, axis=0, keepdims=True), (8, F))


def _bn0_body(y_ref, b_ref, g_ref, bt_ref, h_ref, m_ref):
    t = y_ref[0] + b_ref[0:1, :]
    mu = jnp.mean(t, axis=0, keepdims=True)
    ctr = t - mu
    var = jnp.mean(ctr * ctr, axis=0, keepdims=True)
    hh = jnp.maximum(ctr * lax.rsqrt(var + EPS) * g_ref[0:1, :] + bt_ref[0:1, :],
                     0.0)
    h_ref[...] = hh
    m_ref[...] = jnp.broadcast_to(jnp.mean(hh, axis=0, keepdims=True), (8, F))


def _bn0(y, b8, g8, bt8):
    return pl.pallas_call(
        _bn0_body,
        grid=(NCH,),
        in_specs=[
            pl.BlockSpec((1, N, F), lambda c: (c, 0, 0)),
            pl.BlockSpec((8, F), lambda c: (0, c)),
            pl.BlockSpec((8, F), lambda c: (0, c)),
            pl.BlockSpec((8, F), lambda c: (0, c)),
        ],
        out_specs=[
            pl.BlockSpec((N, F), lambda c: (0, c)),
            pl.BlockSpec((8, F), lambda c: (0, c)),
        ],
        out_shape=[
            jax.ShapeDtypeStruct((N, DH), jnp.float32),
            jax.ShapeDtypeStruct((8, DH), jnp.float32),
        ],
    )(y, b8, g8, bt8)


def _bn(s, y, deg16, b8, g8, bt8):
    return pl.pallas_call(
        _bn_body,
        grid=(NCH,),
        in_specs=[
            pl.BlockSpec((1, ACC_R, F), lambda c: (c, 0, 0)),
            pl.BlockSpec((1, N, F), lambda c: (c, 0, 0)),
            pl.BlockSpec((N, 16), lambda c: (0, 0)),
            pl.BlockSpec((8, F), lambda c: (0, c)),
            pl.BlockSpec((8, F), lambda c: (0, c)),
            pl.BlockSpec((8, F), lambda c: (0, c)),
        ],
        out_specs=[
            pl.BlockSpec((N, F), lambda c: (0, c)),
            pl.BlockSpec((8, F), lambda c: (0, c)),
        ],
        out_shape=[
            jax.ShapeDtypeStruct((N, DH), jnp.float32),
            jax.ShapeDtypeStruct((8, DH), jnp.float32),
        ],
    )(s, y, deg16, b8, g8, bt8)


def kernel(x, edge_index, W0, b0, g0, bt0, W1, b1, g1, bt1,
           W2, b2, g2, bt2, W3, b3, g3, bt3):
    ei = edge_index.astype(jnp.int32)
    pad = EPAD - E
    srcp = jnp.concatenate([ei[0], jnp.zeros((pad,), jnp.int32)])
    dstp = jnp.concatenate([ei[1], jnp.full((pad,), N, jnp.int32)])
    dst16 = dstp.reshape(16, 80, 128)
    dst_a = dstp.reshape(16, 128, 80)
    onesF = jnp.ones((128, F), jnp.float32)
    zerosF = jnp.zeros((80, F), jnp.float32)

    deg16 = _deg(dst16, onesF, zerosF)[:N, :16]

    def p8(v):
        return jnp.broadcast_to(v.reshape(1, DH), (8, DH))

    y0 = _scale(x, deg16)
    s0 = _agg2(y0, srcp, dst_a, zerosF)
    xw0 = _mm0(s0, y0, deg16, W0)
    h, m = _bn0(xw0, p8(b0), p8(g0), p8(bt0))

    for W, b, g, bt in ((W1, b1, g1, bt1), (W2, b2, g2, bt2),
                        (W3, b3, g3, bt3)):
        y = _mm(h, W, deg16)
        s = _agg(y, srcp, dst_a, zerosF)
        h, m = _bn(s, y, deg16, p8(b), p8(g), p8(bt))
    return h, m[0:1]
